# baseline (speedup 1.0000x reference)
import functools

import jax
import jax.numpy as jnp
from jax import lax
from jax.experimental import pallas as pl
from jax.experimental.pallas import tpu as pltpu
from jax.experimental.pallas import tpu_sc as plsc

_H = 768
_FF = 2048
_E = 8
_S = 2048
_T = 256
_FC = 512
_NB = _S // _T + _E
_PS = _NB * _T
_NW = 32
_CHUNK = _S // _NW


def _plan_body(x_ref, gw_ref, pos_ref, be_ref, bv_ref):
    x = x_ref[...]
    gw = gw_ref[...]
    logits = lax.dot_general(
        x.astype(jnp.bfloat16), gw.astype(jnp.bfloat16),
        (((1,), (1,)), ((), ())),
        preferred_element_type=jnp.float32)
    m = jnp.max(logits, axis=1, keepdims=True)
    col = lax.broadcasted_iota(jnp.int32, (_S, _E), 1)
    cand = jnp.where(logits == m, col, _E)
    eid = jnp.min(cand, axis=1, keepdims=True)
    onehot = (col == eid).astype(jnp.float32)

    tri = (lax.broadcasted_iota(jnp.int32, (_T, _T), 0)
           >= lax.broadcasted_iota(jnp.int32, (_T, _T), 1)).astype(jnp.float32)
    chunks = []
    run = jnp.zeros((1, _E), jnp.float32)
    for c in range(_S // _T):
        oh_c = onehot[c * _T:(c + 1) * _T, :]
        w_c = jnp.dot(tri, oh_c, preferred_element_type=jnp.float32)
        chunks.append(w_c + run)
        run = run + w_c[_T - 1:_T, :]
    rank_incl = jnp.concatenate(chunks, axis=0)
    counts = run

    pc = jnp.ceil(counts / _T) * _T
    erow = lax.broadcasted_iota(jnp.int32, (_E, _E), 0)
    ecol = lax.broadcasted_iota(jnp.int32, (_E, _E), 1)
    upper = (erow < ecol).astype(jnp.float32)
    astart = jnp.dot(pc, upper, preferred_element_type=jnp.float32)

    rank_tok = jnp.sum(rank_incl * onehot, axis=1, keepdims=True)
    start_tok = jnp.sum(onehot * astart, axis=1, keepdims=True)
    pos_ref[...] = (start_tok + rank_tok - 1.0).astype(jnp.int32)

    ends = astart + pc
    total = jnp.sum(pc)
    jrow = lax.broadcasted_iota(jnp.int32, (_NB, 1), 0).astype(
        jnp.float32) * _T
    posj = jnp.minimum(jrow, total - _T)
    be = jnp.sum((ends <= posj).astype(jnp.int32), axis=1, keepdims=True)
    be_ref[...] = be
    bv_ref[...] = (jrow < total).astype(jnp.int32)


def _expert_body(be_ref, bv_ref, xs_ref, wg_ref, wu_ref, wd_ref, out_ref,
                 xt_s, acc_s):
    j = pl.program_id(0)
    f = pl.program_id(1)

    @pl.when(bv_ref[j] == 1)
    def _():
        @pl.when(f == 0)
        def _():
            xt_s[...] = xs_ref[...].T
            acc_s[...] = jnp.zeros((_H, _T), jnp.float32)

        xt = xt_s[...]
        g = jnp.dot(wg_ref[0], xt, preferred_element_type=jnp.float32)
        u = jnp.dot(wu_ref[0], xt, preferred_element_type=jnp.float32)
        h = (g / (1.0 + jnp.exp(-g))) * u
        acc_s[...] += jnp.dot(wd_ref[0], h, preferred_element_type=jnp.float32)

        @pl.when(f == _FF // _FC - 1)
        def _():
            out_ref[...] = acc_s[...].T


_plan = pl.pallas_call(
    _plan_body,
    out_shape=(
        jax.ShapeDtypeStruct((_S, 1), jnp.int32),
        jax.ShapeDtypeStruct((_NB, 1), jnp.int32),
        jax.ShapeDtypeStruct((_NB, 1), jnp.int32),
    ),
)

_experts = pl.pallas_call(
    _expert_body,
    grid_spec=pltpu.PrefetchScalarGridSpec(
        num_scalar_prefetch=2,
        grid=(_NB, _FF // _FC),
        in_specs=[
            pl.BlockSpec((_T, _H), lambda j, f, be, bv: (j, 0)),
            pl.BlockSpec((1, _FC, _H), lambda j, f, be, bv: (be[j], f, 0)),
            pl.BlockSpec((1, _FC, _H), lambda j, f, be, bv: (be[j], f, 0)),
            pl.BlockSpec((1, _H, _FC), lambda j, f, be, bv: (be[j], 0, f)),
        ],
        out_specs=pl.BlockSpec((_T, _H), lambda j, f, be, bv: (j, 0)),
        scratch_shapes=[
            pltpu.VMEM((_H, _T), jnp.float32),
            pltpu.VMEM((_H, _T), jnp.float32),
        ],
    ),
    out_shape=jax.ShapeDtypeStruct((_PS, _H), jnp.float32),
)

@functools.cache
def _sc_kernels():
    mesh = plsc.VectorSubcoreMesh(core_axis_name="c", subcore_axis_name="s")
    scratch = [
        pltpu.VMEM((_CHUNK,), jnp.int32),
        pltpu.VMEM((_CHUNK, _H), jnp.float32),
        pltpu.SemaphoreType.DMA,
    ]

    @functools.partial(
        pl.kernel,
        out_type=jax.ShapeDtypeStruct((_PS, _H), jnp.float32),
        mesh=mesh,
        scratch_types=scratch,
    )
    def dispatch(x_hbm, pos_hbm, xs_hbm, idx_v, rows_v, sem):
        wid = lax.axis_index("s") * 2 + lax.axis_index("c")
        base = wid * _CHUNK
        pltpu.sync_copy(pos_hbm.at[pl.ds(base, _CHUNK)], idx_v)
        pltpu.sync_copy(x_hbm.at[pl.ds(base, _CHUNK)], rows_v)
        pltpu.async_copy(rows_v, xs_hbm.at[idx_v], sem).wait()

    @functools.partial(
        pl.kernel,
        out_type=jax.ShapeDtypeStruct((_S, _H), jnp.float32),
        mesh=mesh,
        scratch_types=scratch,
    )
    def combine(po_hbm, pos_hbm, out_hbm, idx_v, rows_v, sem):
        wid = lax.axis_index("s") * 2 + lax.axis_index("c")
        base = wid * _CHUNK
        pltpu.sync_copy(pos_hbm.at[pl.ds(base, _CHUNK)], idx_v)
        pltpu.async_copy(po_hbm.at[idx_v], rows_v, sem).wait()
        pltpu.sync_copy(rows_v, out_hbm.at[pl.ds(base, _CHUNK)])

    return dispatch, combine


@jax.jit
def kernel(x, gate_w, w_gate, w_up, w_down):
    b, s, h = x.shape
    x_flat = x.reshape(s, h)
    dispatch, combine = _sc_kernels()
    pos2, be2, bv2 = _plan(x_flat, gate_w)
    pos = pos2.reshape(s)
    xs = dispatch(x_flat, pos)
    po = _experts(be2.reshape(-1), bv2.reshape(-1), xs, w_gate, w_up, w_down)
    out = combine(po, pos)
    return out.reshape(b, s, h)


# speedup vs baseline: 1.0012x; 1.0012x over previous
import functools

import jax
import jax.numpy as jnp
from jax import lax
from jax.experimental import pallas as pl
from jax.experimental.pallas import tpu as pltpu
from jax.experimental.pallas import tpu_sc as plsc

_H = 768
_FF = 2048
_E = 8
_S = 2048
_T = 256
_FC = 512
_NB = _S // _T + _E
_PS = _NB * _T
_NW = 32
_CHUNK = _S // _NW


def _plan_body(x_ref, gw_ref, pos_ref, nblk_ref):
    x = x_ref[...]
    gw = gw_ref[...]
    logits = lax.dot_general(
        x.astype(jnp.bfloat16), gw.astype(jnp.bfloat16),
        (((1,), (1,)), ((), ())),
        preferred_element_type=jnp.float32)
    m = jnp.max(logits, axis=1, keepdims=True)
    col = lax.broadcasted_iota(jnp.int32, (_S, _E), 1)
    cand = jnp.where(logits == m, col, _E)
    eid = jnp.min(cand, axis=1, keepdims=True)
    onehot = (col == eid).astype(jnp.float32)

    tri = (lax.broadcasted_iota(jnp.int32, (_T, _T), 0)
           >= lax.broadcasted_iota(jnp.int32, (_T, _T), 1)).astype(jnp.float32)
    chunks = []
    run = jnp.zeros((1, _E), jnp.float32)
    for c in range(_S // _T):
        oh_c = onehot[c * _T:(c + 1) * _T, :]
        w_c = jnp.dot(tri, oh_c, preferred_element_type=jnp.float32)
        chunks.append(w_c + run)
        run = run + w_c[_T - 1:_T, :]
    rank_incl = jnp.concatenate(chunks, axis=0)
    counts = run

    pc = jnp.ceil(counts / _T) * _T
    erow = lax.broadcasted_iota(jnp.int32, (_E, _E), 0)
    ecol = lax.broadcasted_iota(jnp.int32, (_E, _E), 1)
    upper = (erow < ecol).astype(jnp.float32)
    astart = jnp.dot(pc, upper, preferred_element_type=jnp.float32)

    rank_tok = jnp.sum(rank_incl * onehot, axis=1, keepdims=True)
    start_tok = jnp.sum(onehot * astart, axis=1, keepdims=True)
    pos_ref[...] = (start_tok + rank_tok - 1.0).astype(jnp.int32)

    nblk_ref[...] = (pc / _T).astype(jnp.int32)


def _expert_body(nblk_ref, xs_hbm, wg_hbm, wu_hbm, wd_hbm, po_hbm,
                 xsb, wgb, wub, wdb, otb, sem_x, sem_w, sem_o):
    def w_copies(e, slot):
        return (
            pltpu.make_async_copy(wg_hbm.at[e], wgb.at[slot], sem_w.at[slot]),
            pltpu.make_async_copy(wu_hbm.at[e], wub.at[slot], sem_w.at[slot]),
            pltpu.make_async_copy(wd_hbm.at[e], wdb.at[slot], sem_w.at[slot]),
        )

    for c in w_copies(0, 0):
        c.start()
    for c in w_copies(1, 1):
        c.start()

    def block_body(slot, g):
        pltpu.make_async_copy(
            xs_hbm.at[pl.ds(g * _T, _T)], xsb, sem_x).start()
        pltpu.make_async_copy(
            xs_hbm.at[pl.ds(g * _T, _T)], xsb, sem_x).wait()
        xt = xsb[...].T
        gt = jnp.dot(wgb[slot], xt, preferred_element_type=jnp.float32)
        u = jnp.dot(wub[slot], xt, preferred_element_type=jnp.float32)
        h = (gt / (1.0 + jnp.exp(-gt))) * u
        ot = jnp.dot(wdb[slot], h, preferred_element_type=jnp.float32)
        otb[...] = ot.T
        pltpu.make_async_copy(
            otb, po_hbm.at[pl.ds(g * _T, _T)], sem_o).start()
        pltpu.make_async_copy(
            otb, po_hbm.at[pl.ds(g * _T, _T)], sem_o).wait()
        return g + 1

    def make_body(slot):
        def body(i, g):
            return block_body(slot, g)
        return body

    g = 0
    for e in range(_E):
        slot = e % 2
        for c in w_copies(e, slot):
            c.wait()
        g = lax.fori_loop(0, nblk_ref[e], make_body(slot), g)
        if e + 2 < _E:
            for c in w_copies(e + 2, slot):
                c.start()


_plan = pl.pallas_call(
    _plan_body,
    out_shape=(
        jax.ShapeDtypeStruct((_S, 1), jnp.int32),
        jax.ShapeDtypeStruct((1, _E), jnp.int32),
    ),
)

_experts = pl.pallas_call(
    _expert_body,
    grid_spec=pltpu.PrefetchScalarGridSpec(
        num_scalar_prefetch=1,
        grid=(1,),
        in_specs=[
            pl.BlockSpec(memory_space=pl.ANY),
            pl.BlockSpec(memory_space=pl.ANY),
            pl.BlockSpec(memory_space=pl.ANY),
            pl.BlockSpec(memory_space=pl.ANY),
        ],
        out_specs=pl.BlockSpec(memory_space=pl.ANY),
        scratch_shapes=[
            pltpu.VMEM((_T, _H), jnp.float32),
            pltpu.VMEM((2, _FF, _H), jnp.float32),
            pltpu.VMEM((2, _FF, _H), jnp.float32),
            pltpu.VMEM((2, _H, _FF), jnp.float32),
            pltpu.VMEM((_T, _H), jnp.float32),
            pltpu.SemaphoreType.DMA,
            pltpu.SemaphoreType.DMA((2,)),
            pltpu.SemaphoreType.DMA,
        ],
    ),
    out_shape=jax.ShapeDtypeStruct((_PS, _H), jnp.float32),
)

@functools.cache
def _sc_kernels():
    mesh = plsc.VectorSubcoreMesh(core_axis_name="c", subcore_axis_name="s")
    scratch = [
        pltpu.VMEM((_CHUNK,), jnp.int32),
        pltpu.VMEM((_CHUNK, _H), jnp.float32),
        pltpu.SemaphoreType.DMA,
    ]

    @functools.partial(
        pl.kernel,
        out_type=jax.ShapeDtypeStruct((_PS, _H), jnp.float32),
        mesh=mesh,
        scratch_types=scratch,
    )
    def dispatch(x_hbm, pos_hbm, xs_hbm, idx_v, rows_v, sem):
        wid = lax.axis_index("s") * 2 + lax.axis_index("c")
        base = wid * _CHUNK
        pltpu.sync_copy(pos_hbm.at[pl.ds(base, _CHUNK)], idx_v)
        pltpu.sync_copy(x_hbm.at[pl.ds(base, _CHUNK)], rows_v)
        pltpu.async_copy(rows_v, xs_hbm.at[idx_v], sem).wait()

    @functools.partial(
        pl.kernel,
        out_type=jax.ShapeDtypeStruct((_S, _H), jnp.float32),
        mesh=mesh,
        scratch_types=scratch,
    )
    def combine(po_hbm, pos_hbm, out_hbm, idx_v, rows_v, sem):
        wid = lax.axis_index("s") * 2 + lax.axis_index("c")
        base = wid * _CHUNK
        pltpu.sync_copy(pos_hbm.at[pl.ds(base, _CHUNK)], idx_v)
        pltpu.async_copy(po_hbm.at[idx_v], rows_v, sem).wait()
        pltpu.sync_copy(rows_v, out_hbm.at[pl.ds(base, _CHUNK)])

    return dispatch, combine


@jax.jit
def kernel(x, gate_w, w_gate, w_up, w_down):
    b, s, h = x.shape
    x_flat = x.reshape(s, h)
    dispatch, combine = _sc_kernels()
    pos2, nblk2 = _plan(x_flat, gate_w)
    pos = pos2.reshape(s)
    xs = dispatch(x_flat, pos)
    po = _experts(nblk2.reshape(-1), xs, w_gate, w_up, w_down)
    out = combine(po, pos)
    return out.reshape(b, s, h)


# speedup vs baseline: 1.4149x; 1.4132x over previous
import functools

import jax
import jax.numpy as jnp
from jax import lax
from jax.experimental import pallas as pl
from jax.experimental.pallas import tpu as pltpu
from jax.experimental.pallas import tpu_sc as plsc

_H = 768
_FF = 2048
_E = 8
_S = 2048
_T = 256
_FC = 512
_NB = _S // _T + _E - 1
_PS = _NB * _T
_NW = 32
_CHUNK = _S // _NW


def _plan_body(x_ref, gw_ref, pos_ref, be_ref, bv_ref):
    x = x_ref[...]
    gw = gw_ref[...]
    logits = lax.dot_general(
        x.astype(jnp.bfloat16), gw.astype(jnp.bfloat16),
        (((1,), (1,)), ((), ())),
        preferred_element_type=jnp.float32)
    m = jnp.max(logits, axis=1, keepdims=True)
    col = lax.broadcasted_iota(jnp.int32, (_S, _E), 1)
    cand = jnp.where(logits == m, col, _E)
    eid = jnp.min(cand, axis=1, keepdims=True)
    onehot = (col == eid).astype(jnp.float32)

    tri = (lax.broadcasted_iota(jnp.int32, (_T, _T), 0)
           >= lax.broadcasted_iota(jnp.int32, (_T, _T), 1)).astype(jnp.float32)
    chunks = []
    run = jnp.zeros((1, _E), jnp.float32)
    for c in range(_S // _T):
        oh_c = onehot[c * _T:(c + 1) * _T, :]
        w_c = jnp.dot(tri, oh_c, preferred_element_type=jnp.float32)
        chunks.append(w_c + run)
        run = run + w_c[_T - 1:_T, :]
    rank_incl = jnp.concatenate(chunks, axis=0)
    counts = run

    pc = jnp.ceil(counts / _T) * _T
    erow = lax.broadcasted_iota(jnp.int32, (_E, _E), 0)
    ecol = lax.broadcasted_iota(jnp.int32, (_E, _E), 1)
    upper = (erow < ecol).astype(jnp.float32)
    astart = jnp.dot(pc, upper, preferred_element_type=jnp.float32)

    rank_tok = jnp.sum(rank_incl * onehot, axis=1, keepdims=True)
    start_tok = jnp.sum(onehot * astart, axis=1, keepdims=True)
    pos_ref[...] = (start_tok + rank_tok - 1.0).astype(jnp.int32)

    ends = astart + pc
    total = jnp.sum(pc)
    jrow = lax.broadcasted_iota(jnp.int32, (_NB, 1), 0).astype(
        jnp.float32) * _T
    posj = jnp.minimum(jrow, total - _T)
    be = jnp.sum((ends <= posj).astype(jnp.int32), axis=1, keepdims=True)
    be_ref[...] = be
    bv_ref[...] = (jrow < total).astype(jnp.int32)


def _expert_body(be_ref, bv_ref, xs_ref, wg_ref, wu_ref, wd_ref, out_ref):
    j = pl.program_id(0)

    @pl.when(bv_ref[j] == 1)
    def _():
        xb = xs_ref[...]
        nt = (((1,), (1,)), ((), ()))
        g = lax.dot_general(xb, wg_ref[0], nt,
                            preferred_element_type=jnp.float32)
        u = lax.dot_general(xb, wu_ref[0], nt,
                            preferred_element_type=jnp.float32)
        h = (g / (1.0 + jnp.exp(-g))) * u
        out_ref[...] = lax.dot_general(h, wd_ref[0], nt,
                                       preferred_element_type=jnp.float32)


_plan = pl.pallas_call(
    _plan_body,
    out_shape=(
        jax.ShapeDtypeStruct((_S, 1), jnp.int32),
        jax.ShapeDtypeStruct((_NB, 1), jnp.int32),
        jax.ShapeDtypeStruct((_NB, 1), jnp.int32),
    ),
)

_experts = pl.pallas_call(
    _expert_body,
    grid_spec=pltpu.PrefetchScalarGridSpec(
        num_scalar_prefetch=2,
        grid=(_NB,),
        in_specs=[
            pl.BlockSpec((_T, _H), lambda j, be, bv: (j, 0)),
            pl.BlockSpec((1, _FF, _H), lambda j, be, bv: (be[j], 0, 0)),
            pl.BlockSpec((1, _FF, _H), lambda j, be, bv: (be[j], 0, 0)),
            pl.BlockSpec((1, _H, _FF), lambda j, be, bv: (be[j], 0, 0)),
        ],
        out_specs=pl.BlockSpec((_T, _H), lambda j, be, bv: (j, 0)),
    ),
    out_shape=jax.ShapeDtypeStruct((_PS, _H), jnp.float32),
)

@functools.cache
def _sc_kernels():
    mesh = plsc.VectorSubcoreMesh(core_axis_name="c", subcore_axis_name="s")
    scratch = [
        pltpu.VMEM((_CHUNK,), jnp.int32),
        pltpu.VMEM((_CHUNK, _H), jnp.float32),
        pltpu.SemaphoreType.DMA,
    ]

    @functools.partial(
        pl.kernel,
        out_type=jax.ShapeDtypeStruct((_PS, _H), jnp.float32),
        mesh=mesh,
        scratch_types=scratch,
    )
    def dispatch(x_hbm, pos_hbm, xs_hbm, idx_v, rows_v, sem):
        wid = lax.axis_index("s") * 2 + lax.axis_index("c")
        base = wid * _CHUNK
        pltpu.sync_copy(pos_hbm.at[pl.ds(base, _CHUNK)], idx_v)
        pltpu.sync_copy(x_hbm.at[pl.ds(base, _CHUNK)], rows_v)
        pltpu.async_copy(rows_v, xs_hbm.at[idx_v], sem).wait()

    @functools.partial(
        pl.kernel,
        out_type=jax.ShapeDtypeStruct((_S, _H), jnp.float32),
        mesh=mesh,
        scratch_types=scratch,
    )
    def combine(po_hbm, pos_hbm, out_hbm, idx_v, rows_v, sem):
        wid = lax.axis_index("s") * 2 + lax.axis_index("c")
        base = wid * _CHUNK
        pltpu.sync_copy(pos_hbm.at[pl.ds(base, _CHUNK)], idx_v)
        pltpu.async_copy(po_hbm.at[idx_v], rows_v, sem).wait()
        pltpu.sync_copy(rows_v, out_hbm.at[pl.ds(base, _CHUNK)])

    return dispatch, combine


@jax.jit
def kernel(x, gate_w, w_gate, w_up, w_down):
    b, s, h = x.shape
    x_flat = x.reshape(s, h)
    dispatch, combine = _sc_kernels()
    pos2, be2, bv2 = _plan(x_flat, gate_w)
    pos = pos2.reshape(s)
    xs = dispatch(x_flat, pos)
    po = _experts(be2.reshape(-1), bv2.reshape(-1), xs, w_gate, w_up, w_down)
    out = combine(po, pos)
    return out.reshape(b, s, h)


# speedup vs baseline: 1.5245x; 1.0775x over previous
import functools

import jax
import jax.numpy as jnp
from jax import lax
from jax.experimental import pallas as pl
from jax.experimental.pallas import tpu as pltpu
from jax.experimental.pallas import tpu_sc as plsc

_H = 768
_FF = 2048
_E = 8
_S = 2048
_T = 512
_FC = 512
_NB = _S // _T + _E - 1
_PS = _NB * _T
_NW = 32
_CHUNK = _S // _NW


def _plan_body(x_ref, gw_ref, pos_ref, be_ref, bv_ref):
    x = x_ref[...]
    gw = gw_ref[...]
    logits = lax.dot_general(
        x.astype(jnp.bfloat16), gw.astype(jnp.bfloat16),
        (((1,), (1,)), ((), ())),
        preferred_element_type=jnp.float32)
    m = jnp.max(logits, axis=1, keepdims=True)
    col = lax.broadcasted_iota(jnp.int32, (_S, _E), 1)
    cand = jnp.where(logits == m, col, _E)
    eid = jnp.min(cand, axis=1, keepdims=True)
    onehot = (col == eid).astype(jnp.float32)

    tri = (lax.broadcasted_iota(jnp.int32, (_T, _T), 0)
           >= lax.broadcasted_iota(jnp.int32, (_T, _T), 1)).astype(jnp.float32)
    chunks = []
    run = jnp.zeros((1, _E), jnp.float32)
    for c in range(_S // _T):
        oh_c = onehot[c * _T:(c + 1) * _T, :]
        w_c = jnp.dot(tri, oh_c, preferred_element_type=jnp.float32)
        chunks.append(w_c + run)
        run = run + w_c[_T - 1:_T, :]
    rank_incl = jnp.concatenate(chunks, axis=0)
    counts = run

    pc = jnp.ceil(counts / _T) * _T
    erow = lax.broadcasted_iota(jnp.int32, (_E, _E), 0)
    ecol = lax.broadcasted_iota(jnp.int32, (_E, _E), 1)
    upper = (erow < ecol).astype(jnp.float32)
    astart = jnp.dot(pc, upper, preferred_element_type=jnp.float32)

    rank_tok = jnp.sum(rank_incl * onehot, axis=1, keepdims=True)
    start_tok = jnp.sum(onehot * astart, axis=1, keepdims=True)
    pos_ref[...] = (start_tok + rank_tok - 1.0).astype(jnp.int32)

    ends = astart + pc
    total = jnp.sum(pc)
    jrow = lax.broadcasted_iota(jnp.int32, (_NB, 1), 0).astype(
        jnp.float32) * _T
    posj = jnp.minimum(jrow, total - _T)
    be = jnp.sum((ends <= posj).astype(jnp.int32), axis=1, keepdims=True)
    be_ref[...] = be
    bv_ref[...] = (jrow < total).astype(jnp.int32)


def _expert_body(be_ref, bv_ref, xs_ref, wg_ref, wu_ref, wd_ref, out_ref):
    j = pl.program_id(0)

    @pl.when(bv_ref[j] == 1)
    def _():
        xb = xs_ref[...]
        nt = (((1,), (1,)), ((), ()))
        g = lax.dot_general(xb, wg_ref[0], nt,
                            preferred_element_type=jnp.float32)
        u = lax.dot_general(xb, wu_ref[0], nt,
                            preferred_element_type=jnp.float32)
        h = (g / (1.0 + jnp.exp(-g))) * u
        out_ref[...] = lax.dot_general(h, wd_ref[0], nt,
                                       preferred_element_type=jnp.float32)


_plan = pl.pallas_call(
    _plan_body,
    out_shape=(
        jax.ShapeDtypeStruct((_S, 1), jnp.int32),
        jax.ShapeDtypeStruct((_NB, 1), jnp.int32),
        jax.ShapeDtypeStruct((_NB, 1), jnp.int32),
    ),
)

_experts = pl.pallas_call(
    _expert_body,
    grid_spec=pltpu.PrefetchScalarGridSpec(
        num_scalar_prefetch=2,
        grid=(_NB,),
        in_specs=[
            pl.BlockSpec((_T, _H), lambda j, be, bv: (j, 0)),
            pl.BlockSpec((1, _FF, _H), lambda j, be, bv: (be[j], 0, 0)),
            pl.BlockSpec((1, _FF, _H), lambda j, be, bv: (be[j], 0, 0)),
            pl.BlockSpec((1, _H, _FF), lambda j, be, bv: (be[j], 0, 0)),
        ],
        out_specs=pl.BlockSpec((_T, _H), lambda j, be, bv: (j, 0)),
    ),
    out_shape=jax.ShapeDtypeStruct((_PS, _H), jnp.float32),
)

@functools.cache
def _sc_kernels():
    mesh = plsc.VectorSubcoreMesh(core_axis_name="c", subcore_axis_name="s")
    scratch = [
        pltpu.VMEM((_CHUNK,), jnp.int32),
        pltpu.VMEM((_CHUNK, _H), jnp.float32),
        pltpu.SemaphoreType.DMA,
    ]

    @functools.partial(
        pl.kernel,
        out_type=jax.ShapeDtypeStruct((_PS, _H), jnp.float32),
        mesh=mesh,
        scratch_types=scratch,
    )
    def dispatch(x_hbm, pos_hbm, xs_hbm, idx_v, rows_v, sem):
        wid = lax.axis_index("s") * 2 + lax.axis_index("c")
        base = wid * _CHUNK
        pltpu.sync_copy(pos_hbm.at[pl.ds(base, _CHUNK)], idx_v)
        pltpu.sync_copy(x_hbm.at[pl.ds(base, _CHUNK)], rows_v)
        pltpu.async_copy(rows_v, xs_hbm.at[idx_v], sem).wait()

    @functools.partial(
        pl.kernel,
        out_type=jax.ShapeDtypeStruct((_S, _H), jnp.float32),
        mesh=mesh,
        scratch_types=scratch,
    )
    def combine(po_hbm, pos_hbm, out_hbm, idx_v, rows_v, sem):
        wid = lax.axis_index("s") * 2 + lax.axis_index("c")
        base = wid * _CHUNK
        pltpu.sync_copy(pos_hbm.at[pl.ds(base, _CHUNK)], idx_v)
        pltpu.async_copy(po_hbm.at[idx_v], rows_v, sem).wait()
        pltpu.sync_copy(rows_v, out_hbm.at[pl.ds(base, _CHUNK)])

    return dispatch, combine


@jax.jit
def kernel(x, gate_w, w_gate, w_up, w_down):
    b, s, h = x.shape
    x_flat = x.reshape(s, h)
    dispatch, combine = _sc_kernels()
    pos2, be2, bv2 = _plan(x_flat, gate_w)
    pos = pos2.reshape(s)
    xs = dispatch(x_flat, pos)
    po = _experts(be2.reshape(-1), bv2.reshape(-1), xs, w_gate, w_up, w_down)
    out = combine(po, pos)
    return out.reshape(b, s, h)


# speedup vs baseline: 1.5865x; 1.0407x over previous
import functools

import jax
import jax.numpy as jnp
from jax import lax
from jax.experimental import pallas as pl
from jax.experimental.pallas import tpu as pltpu
from jax.experimental.pallas import tpu_sc as plsc

_H = 768
_FF = 2048
_E = 8
_S = 2048
_T = 384
_FC = 512
_NB = (_S + _T - 1) // _T + _E - 1
_PS = _NB * _T
_CS = 256
_NW = 32
_CHUNK = _S // _NW


def _plan_body(x_ref, gw_ref, pos_ref, be_ref, bv_ref):
    x = x_ref[...]
    gw = gw_ref[...]
    logits = lax.dot_general(
        x.astype(jnp.bfloat16), gw.astype(jnp.bfloat16),
        (((1,), (1,)), ((), ())),
        preferred_element_type=jnp.float32)
    m = jnp.max(logits, axis=1, keepdims=True)
    col = lax.broadcasted_iota(jnp.int32, (_S, _E), 1)
    cand = jnp.where(logits == m, col, _E)
    eid = jnp.min(cand, axis=1, keepdims=True)
    onehot = (col == eid).astype(jnp.float32)

    tri = (lax.broadcasted_iota(jnp.int32, (_CS, _CS), 0)
           >= lax.broadcasted_iota(jnp.int32, (_CS, _CS), 1)).astype(jnp.float32)
    chunks = []
    run = jnp.zeros((1, _E), jnp.float32)
    for c in range(_S // _CS):
        oh_c = onehot[c * _CS:(c + 1) * _CS, :]
        w_c = jnp.dot(tri, oh_c, preferred_element_type=jnp.float32)
        chunks.append(w_c + run)
        run = run + w_c[_CS - 1:_CS, :]
    rank_incl = jnp.concatenate(chunks, axis=0)
    counts = run

    pc = jnp.ceil(counts / _T) * _T
    erow = lax.broadcasted_iota(jnp.int32, (_E, _E), 0)
    ecol = lax.broadcasted_iota(jnp.int32, (_E, _E), 1)
    upper = (erow < ecol).astype(jnp.float32)
    astart = jnp.dot(pc, upper, preferred_element_type=jnp.float32)

    rank_tok = jnp.sum(rank_incl * onehot, axis=1, keepdims=True)
    start_tok = jnp.sum(onehot * astart, axis=1, keepdims=True)
    pos_ref[...] = (start_tok + rank_tok - 1.0).astype(jnp.int32)

    ends = astart + pc
    total = jnp.sum(pc)
    jrow = lax.broadcasted_iota(jnp.int32, (_NB, 1), 0).astype(
        jnp.float32) * _T
    posj = jnp.minimum(jrow, total - _T)
    be = jnp.sum((ends <= posj).astype(jnp.int32), axis=1, keepdims=True)
    be_ref[...] = be
    bv_ref[...] = (jrow < total).astype(jnp.int32)


def _expert_body(be_ref, bv_ref, xs_ref, wg_ref, wu_ref, wd_ref, out_ref):
    j = pl.program_id(0)

    @pl.when(bv_ref[j] == 1)
    def _():
        xb = xs_ref[...]
        nt = (((1,), (1,)), ((), ()))
        g = lax.dot_general(xb, wg_ref[0], nt,
                            preferred_element_type=jnp.float32)
        u = lax.dot_general(xb, wu_ref[0], nt,
                            preferred_element_type=jnp.float32)
        h = (g / (1.0 + jnp.exp(-g))) * u
        out_ref[...] = lax.dot_general(h, wd_ref[0], nt,
                                       preferred_element_type=jnp.float32)


_plan = pl.pallas_call(
    _plan_body,
    out_shape=(
        jax.ShapeDtypeStruct((_S, 1), jnp.int32),
        jax.ShapeDtypeStruct((_NB, 1), jnp.int32),
        jax.ShapeDtypeStruct((_NB, 1), jnp.int32),
    ),
)

_experts = pl.pallas_call(
    _expert_body,
    grid_spec=pltpu.PrefetchScalarGridSpec(
        num_scalar_prefetch=2,
        grid=(_NB,),
        in_specs=[
            pl.BlockSpec((_T, _H), lambda j, be, bv: (j, 0)),
            pl.BlockSpec((1, _FF, _H), lambda j, be, bv: (be[j], 0, 0)),
            pl.BlockSpec((1, _FF, _H), lambda j, be, bv: (be[j], 0, 0)),
            pl.BlockSpec((1, _H, _FF), lambda j, be, bv: (be[j], 0, 0)),
        ],
        out_specs=pl.BlockSpec((_T, _H), lambda j, be, bv: (j, 0)),
    ),
    out_shape=jax.ShapeDtypeStruct((_PS, _H), jnp.float32),
)

@functools.cache
def _sc_kernels():
    mesh = plsc.VectorSubcoreMesh(core_axis_name="c", subcore_axis_name="s")
    scratch = [
        pltpu.VMEM((_CHUNK,), jnp.int32),
        pltpu.VMEM((_CHUNK, _H), jnp.float32),
        pltpu.SemaphoreType.DMA,
    ]

    @functools.partial(
        pl.kernel,
        out_type=jax.ShapeDtypeStruct((_PS, _H), jnp.float32),
        mesh=mesh,
        scratch_types=scratch,
    )
    def dispatch(x_hbm, pos_hbm, xs_hbm, idx_v, rows_v, sem):
        wid = lax.axis_index("s") * 2 + lax.axis_index("c")
        base = wid * _CHUNK
        pltpu.sync_copy(pos_hbm.at[pl.ds(base, _CHUNK)], idx_v)
        pltpu.sync_copy(x_hbm.at[pl.ds(base, _CHUNK)], rows_v)
        pltpu.async_copy(rows_v, xs_hbm.at[idx_v], sem).wait()

    @functools.partial(
        pl.kernel,
        out_type=jax.ShapeDtypeStruct((_S, _H), jnp.float32),
        mesh=mesh,
        scratch_types=scratch,
    )
    def combine(po_hbm, pos_hbm, out_hbm, idx_v, rows_v, sem):
        wid = lax.axis_index("s") * 2 + lax.axis_index("c")
        base = wid * _CHUNK
        pltpu.sync_copy(pos_hbm.at[pl.ds(base, _CHUNK)], idx_v)
        pltpu.async_copy(po_hbm.at[idx_v], rows_v, sem).wait()
        pltpu.sync_copy(rows_v, out_hbm.at[pl.ds(base, _CHUNK)])

    return dispatch, combine


@jax.jit
def kernel(x, gate_w, w_gate, w_up, w_down):
    b, s, h = x.shape
    x_flat = x.reshape(s, h)
    dispatch, combine = _sc_kernels()
    pos2, be2, bv2 = _plan(x_flat, gate_w)
    pos = pos2.reshape(s)
    xs = dispatch(x_flat, pos)
    po = _experts(be2.reshape(-1), bv2.reshape(-1), xs, w_gate, w_up, w_down)
    out = combine(po, pos)
    return out.reshape(b, s, h)


# speedup vs baseline: 1.5902x; 1.0023x over previous
import functools

import jax
import jax.numpy as jnp
from jax import lax
from jax.experimental import pallas as pl
from jax.experimental.pallas import tpu as pltpu
from jax.experimental.pallas import tpu_sc as plsc

_H = 768
_FF = 2048
_E = 8
_S = 2048
_T = 384
_FC = 512
_NB = (_S + _T - 1) // _T + _E - 1
_PS = _NB * _T
_CS = 256
_NW = 32
_CHUNK = _S // _NW


def _plan_body(x_ref, gw_ref, pos_ref, be_ref, bv_ref):
    x = x_ref[...]
    gw = gw_ref[...]
    logits = lax.dot_general(
        x.astype(jnp.bfloat16), gw.astype(jnp.bfloat16),
        (((1,), (1,)), ((), ())),
        preferred_element_type=jnp.float32)
    m = jnp.max(logits, axis=1, keepdims=True)
    col = lax.broadcasted_iota(jnp.int32, (_S, _E), 1)
    cand = jnp.where(logits == m, col, _E)
    eid = jnp.min(cand, axis=1, keepdims=True)
    onehot = (col == eid).astype(jnp.float32)

    tri = (lax.broadcasted_iota(jnp.int32, (_CS, _CS), 0)
           >= lax.broadcasted_iota(jnp.int32, (_CS, _CS), 1)).astype(jnp.float32)
    chunks = []
    run = jnp.zeros((1, _E), jnp.float32)
    for c in range(_S // _CS):
        oh_c = onehot[c * _CS:(c + 1) * _CS, :]
        w_c = jnp.dot(tri, oh_c, preferred_element_type=jnp.float32)
        chunks.append(w_c + run)
        run = run + w_c[_CS - 1:_CS, :]
    rank_incl = jnp.concatenate(chunks, axis=0)
    counts = run

    pc = jnp.ceil(counts / _T) * _T
    erow = lax.broadcasted_iota(jnp.int32, (_E, _E), 0)
    ecol = lax.broadcasted_iota(jnp.int32, (_E, _E), 1)
    upper = (erow < ecol).astype(jnp.float32)
    astart = jnp.dot(pc, upper, preferred_element_type=jnp.float32)

    rank_tok = jnp.sum(rank_incl * onehot, axis=1, keepdims=True)
    start_tok = jnp.sum(onehot * astart, axis=1, keepdims=True)
    pos_ref[...] = (start_tok + rank_tok - 1.0).astype(jnp.int32)

    ends = astart + pc
    total = jnp.sum(pc)
    jrow = lax.broadcasted_iota(jnp.int32, (_NB, 1), 0).astype(
        jnp.float32) * _T
    posj = jnp.minimum(jrow, total - _T)
    be = jnp.sum((ends <= posj).astype(jnp.int32), axis=1, keepdims=True)
    be_ref[...] = be
    bv_ref[...] = (jrow < total).astype(jnp.int32)


def _expert_body(be_ref, bv_ref, xs_ref, wg_ref, wu_ref, wd_ref, out_ref):
    j = pl.program_id(0)

    @pl.when(bv_ref[j] == 1)
    def _():
        xb = xs_ref[...].astype(jnp.bfloat16)
        nt = (((1,), (1,)), ((), ()))
        g = lax.dot_general(xb, wg_ref[0].astype(jnp.bfloat16), nt,
                            preferred_element_type=jnp.float32)
        u = lax.dot_general(xb, wu_ref[0].astype(jnp.bfloat16), nt,
                            preferred_element_type=jnp.float32)
        h = (g / (1.0 + jnp.exp(-g))) * u
        out_ref[...] = lax.dot_general(
            h.astype(jnp.bfloat16), wd_ref[0].astype(jnp.bfloat16), nt,
            preferred_element_type=jnp.float32)


_plan = pl.pallas_call(
    _plan_body,
    out_shape=(
        jax.ShapeDtypeStruct((_S, 1), jnp.int32),
        jax.ShapeDtypeStruct((_NB, 1), jnp.int32),
        jax.ShapeDtypeStruct((_NB, 1), jnp.int32),
    ),
)

_experts = pl.pallas_call(
    _expert_body,
    grid_spec=pltpu.PrefetchScalarGridSpec(
        num_scalar_prefetch=2,
        grid=(_NB,),
        in_specs=[
            pl.BlockSpec((_T, _H), lambda j, be, bv: (j, 0)),
            pl.BlockSpec((1, _FF, _H), lambda j, be, bv: (be[j], 0, 0)),
            pl.BlockSpec((1, _FF, _H), lambda j, be, bv: (be[j], 0, 0)),
            pl.BlockSpec((1, _H, _FF), lambda j, be, bv: (be[j], 0, 0)),
        ],
        out_specs=pl.BlockSpec((_T, _H), lambda j, be, bv: (j, 0)),
    ),
    out_shape=jax.ShapeDtypeStruct((_PS, _H), jnp.float32),
)

@functools.cache
def _sc_kernels():
    mesh = plsc.VectorSubcoreMesh(core_axis_name="c", subcore_axis_name="s")
    scratch = [
        pltpu.VMEM((_CHUNK,), jnp.int32),
        pltpu.VMEM((_CHUNK, _H), jnp.float32),
        pltpu.SemaphoreType.DMA,
    ]

    @functools.partial(
        pl.kernel,
        out_type=jax.ShapeDtypeStruct((_PS, _H), jnp.float32),
        mesh=mesh,
        scratch_types=scratch,
    )
    def dispatch(x_hbm, pos_hbm, xs_hbm, idx_v, rows_v, sem):
        wid = lax.axis_index("s") * 2 + lax.axis_index("c")
        base = wid * _CHUNK
        pltpu.sync_copy(pos_hbm.at[pl.ds(base, _CHUNK)], idx_v)
        pltpu.sync_copy(x_hbm.at[pl.ds(base, _CHUNK)], rows_v)
        pltpu.async_copy(rows_v, xs_hbm.at[idx_v], sem).wait()

    @functools.partial(
        pl.kernel,
        out_type=jax.ShapeDtypeStruct((_S, _H), jnp.float32),
        mesh=mesh,
        scratch_types=scratch,
    )
    def combine(po_hbm, pos_hbm, out_hbm, idx_v, rows_v, sem):
        wid = lax.axis_index("s") * 2 + lax.axis_index("c")
        base = wid * _CHUNK
        pltpu.sync_copy(pos_hbm.at[pl.ds(base, _CHUNK)], idx_v)
        pltpu.async_copy(po_hbm.at[idx_v], rows_v, sem).wait()
        pltpu.sync_copy(rows_v, out_hbm.at[pl.ds(base, _CHUNK)])

    return dispatch, combine


@jax.jit
def kernel(x, gate_w, w_gate, w_up, w_down):
    b, s, h = x.shape
    x_flat = x.reshape(s, h)
    dispatch, combine = _sc_kernels()
    pos2, be2, bv2 = _plan(x_flat, gate_w)
    pos = pos2.reshape(s)
    xs = dispatch(x_flat, pos)
    po = _experts(be2.reshape(-1), bv2.reshape(-1), xs, w_gate, w_up, w_down)
    out = combine(po, pos)
    return out.reshape(b, s, h)


# speedup vs baseline: 1.5915x; 1.0008x over previous
import functools

import jax
import jax.numpy as jnp
from jax import lax
from jax.experimental import pallas as pl
from jax.experimental.pallas import tpu as pltpu
from jax.experimental.pallas import tpu_sc as plsc

_H = 768
_FF = 2048
_E = 8
_S = 2048
_T = 320
_FC = 512
_NB = (_S + _T - 1) // _T + _E - 1
_PS = _NB * _T
_CS = 256
_NW = 32
_CHUNK = _S // _NW


def _plan_body(x_ref, gw_ref, pos_ref, be_ref, bv_ref):
    x = x_ref[...]
    gw = gw_ref[...]
    logits = lax.dot_general(
        x.astype(jnp.bfloat16), gw.astype(jnp.bfloat16),
        (((1,), (1,)), ((), ())),
        preferred_element_type=jnp.float32)
    m = jnp.max(logits, axis=1, keepdims=True)
    col = lax.broadcasted_iota(jnp.int32, (_S, _E), 1)
    cand = jnp.where(logits == m, col, _E)
    eid = jnp.min(cand, axis=1, keepdims=True)
    onehot = (col == eid).astype(jnp.float32)

    tri = (lax.broadcasted_iota(jnp.int32, (_CS, _CS), 0)
           >= lax.broadcasted_iota(jnp.int32, (_CS, _CS), 1)).astype(jnp.float32)
    chunks = []
    run = jnp.zeros((1, _E), jnp.float32)
    for c in range(_S // _CS):
        oh_c = onehot[c * _CS:(c + 1) * _CS, :]
        w_c = jnp.dot(tri, oh_c, preferred_element_type=jnp.float32)
        chunks.append(w_c + run)
        run = run + w_c[_CS - 1:_CS, :]
    rank_incl = jnp.concatenate(chunks, axis=0)
    counts = run

    pc = jnp.ceil(counts / _T) * _T
    erow = lax.broadcasted_iota(jnp.int32, (_E, _E), 0)
    ecol = lax.broadcasted_iota(jnp.int32, (_E, _E), 1)
    upper = (erow < ecol).astype(jnp.float32)
    astart = jnp.dot(pc, upper, preferred_element_type=jnp.float32)

    rank_tok = jnp.sum(rank_incl * onehot, axis=1, keepdims=True)
    start_tok = jnp.sum(onehot * astart, axis=1, keepdims=True)
    pos_ref[...] = (start_tok + rank_tok - 1.0).astype(jnp.int32)

    ends = astart + pc
    total = jnp.sum(pc)
    jrow = lax.broadcasted_iota(jnp.int32, (_NB, 1), 0).astype(
        jnp.float32) * _T
    posj = jnp.minimum(jrow, total - _T)
    be = jnp.sum((ends <= posj).astype(jnp.int32), axis=1, keepdims=True)
    be_ref[...] = be
    bv_ref[...] = (jrow < total).astype(jnp.int32)


def _expert_body(be_ref, bv_ref, xs_ref, wg_ref, wu_ref, wd_ref, out_ref):
    j = pl.program_id(0)

    @pl.when(bv_ref[j] == 1)
    def _():
        xb = xs_ref[...].astype(jnp.bfloat16)
        nt = (((1,), (1,)), ((), ()))
        g = lax.dot_general(xb, wg_ref[0].astype(jnp.bfloat16), nt,
                            preferred_element_type=jnp.float32)
        u = lax.dot_general(xb, wu_ref[0].astype(jnp.bfloat16), nt,
                            preferred_element_type=jnp.float32)
        h = (g / (1.0 + jnp.exp(-g))) * u
        out_ref[...] = lax.dot_general(
            h.astype(jnp.bfloat16), wd_ref[0].astype(jnp.bfloat16), nt,
            preferred_element_type=jnp.float32)


_plan = pl.pallas_call(
    _plan_body,
    out_shape=(
        jax.ShapeDtypeStruct((_S, 1), jnp.int32),
        jax.ShapeDtypeStruct((_NB, 1), jnp.int32),
        jax.ShapeDtypeStruct((_NB, 1), jnp.int32),
    ),
)

_experts = pl.pallas_call(
    _expert_body,
    grid_spec=pltpu.PrefetchScalarGridSpec(
        num_scalar_prefetch=2,
        grid=(_NB,),
        in_specs=[
            pl.BlockSpec((_T, _H), lambda j, be, bv: (j, 0)),
            pl.BlockSpec((1, _FF, _H), lambda j, be, bv: (be[j], 0, 0)),
            pl.BlockSpec((1, _FF, _H), lambda j, be, bv: (be[j], 0, 0)),
            pl.BlockSpec((1, _H, _FF), lambda j, be, bv: (be[j], 0, 0)),
        ],
        out_specs=pl.BlockSpec((_T, _H), lambda j, be, bv: (j, 0)),
    ),
    out_shape=jax.ShapeDtypeStruct((_PS, _H), jnp.float32),
)

@functools.cache
def _sc_kernels():
    mesh = plsc.VectorSubcoreMesh(core_axis_name="c", subcore_axis_name="s")
    scratch = [
        pltpu.VMEM((_CHUNK,), jnp.int32),
        pltpu.VMEM((_CHUNK, _H), jnp.float32),
        pltpu.SemaphoreType.DMA,
    ]

    @functools.partial(
        pl.kernel,
        out_type=jax.ShapeDtypeStruct((_PS, _H), jnp.float32),
        mesh=mesh,
        scratch_types=scratch,
    )
    def dispatch(x_hbm, pos_hbm, xs_hbm, idx_v, rows_v, sem):
        wid = lax.axis_index("s") * 2 + lax.axis_index("c")
        base = wid * _CHUNK
        pltpu.sync_copy(pos_hbm.at[pl.ds(base, _CHUNK)], idx_v)
        pltpu.sync_copy(x_hbm.at[pl.ds(base, _CHUNK)], rows_v)
        pltpu.async_copy(rows_v, xs_hbm.at[idx_v], sem).wait()

    @functools.partial(
        pl.kernel,
        out_type=jax.ShapeDtypeStruct((_S, _H), jnp.float32),
        mesh=mesh,
        scratch_types=scratch,
    )
    def combine(po_hbm, pos_hbm, out_hbm, idx_v, rows_v, sem):
        wid = lax.axis_index("s") * 2 + lax.axis_index("c")
        base = wid * _CHUNK
        pltpu.sync_copy(pos_hbm.at[pl.ds(base, _CHUNK)], idx_v)
        pltpu.async_copy(po_hbm.at[idx_v], rows_v, sem).wait()
        pltpu.sync_copy(rows_v, out_hbm.at[pl.ds(base, _CHUNK)])

    return dispatch, combine


@jax.jit
def kernel(x, gate_w, w_gate, w_up, w_down):
    b, s, h = x.shape
    x_flat = x.reshape(s, h)
    dispatch, combine = _sc_kernels()
    pos2, be2, bv2 = _plan(x_flat, gate_w)
    pos = pos2.reshape(s)
    xs = dispatch(x_flat, pos)
    po = _experts(be2.reshape(-1), bv2.reshape(-1), xs, w_gate, w_up, w_down)
    out = combine(po, pos)
    return out.reshape(b, s, h)


# speedup vs baseline: 1.5920x; 1.0003x over previous
import functools

import jax
import jax.numpy as jnp
from jax import lax
from jax.experimental import pallas as pl
from jax.experimental.pallas import tpu as pltpu
from jax.experimental.pallas import tpu_sc as plsc

_H = 768
_FF = 2048
_E = 8
_S = 2048
_T = 320
_NB = (_S + _T - 1) // _T + _E - 1
_PS = _NB * _T
_CS = 256
_NW = 32
_CHUNK = _S // _NW


def _plan_body(x_ref, gw_ref, pos_ref, be_ref, bv_ref):
    x = x_ref[...]
    gw = gw_ref[...]
    logits = lax.dot_general(
        x.astype(jnp.bfloat16), gw.astype(jnp.bfloat16),
        (((1,), (1,)), ((), ())),
        preferred_element_type=jnp.float32)
    m = jnp.max(logits, axis=1, keepdims=True)
    col = lax.broadcasted_iota(jnp.int32, (_S, _E), 1)
    cand = jnp.where(logits == m, col, _E)
    eid = jnp.min(cand, axis=1, keepdims=True)
    onehot = (col == eid).astype(jnp.float32)

    tri = (lax.broadcasted_iota(jnp.int32, (_CS, _CS), 0)
           >= lax.broadcasted_iota(jnp.int32, (_CS, _CS), 1)).astype(jnp.float32)
    chunks = []
    run = jnp.zeros((1, _E), jnp.float32)
    for c in range(_S // _CS):
        oh_c = onehot[c * _CS:(c + 1) * _CS, :]
        w_c = jnp.dot(tri, oh_c, preferred_element_type=jnp.float32)
        chunks.append(w_c + run)
        run = run + w_c[_CS - 1:_CS, :]
    rank_incl = jnp.concatenate(chunks, axis=0)
    counts = run

    pc = jnp.ceil(counts / _T) * _T
    erow = lax.broadcasted_iota(jnp.int32, (_E, _E), 0)
    ecol = lax.broadcasted_iota(jnp.int32, (_E, _E), 1)
    upper = (erow < ecol).astype(jnp.float32)
    astart = jnp.dot(pc, upper, preferred_element_type=jnp.float32)

    rank_tok = jnp.sum(rank_incl * onehot, axis=1, keepdims=True)
    start_tok = jnp.sum(onehot * astart, axis=1, keepdims=True)
    pos_ref[...] = (start_tok + rank_tok - 1.0).astype(jnp.int32)

    ends = astart + pc
    total = jnp.sum(pc)
    jrow = lax.broadcasted_iota(jnp.int32, (_NB, 1), 0).astype(
        jnp.float32) * _T
    posj = jnp.minimum(jrow, total - _T)
    be = jnp.sum((ends <= posj).astype(jnp.int32), axis=1, keepdims=True)
    be_ref[...] = be
    bv_ref[...] = (jrow < total).astype(jnp.int32)


def _expert_body(be_ref, bv_ref, xs_ref, wg_ref, wu_ref, wd_ref, out_ref):
    j = pl.program_id(0)

    @pl.when(bv_ref[j] == 1)
    def _():
        xb = xs_ref[...].astype(jnp.bfloat16)
        nt = (((1,), (1,)), ((), ()))
        g = lax.dot_general(xb, wg_ref[0].astype(jnp.bfloat16), nt,
                            preferred_element_type=jnp.float32)
        u = lax.dot_general(xb, wu_ref[0].astype(jnp.bfloat16), nt,
                            preferred_element_type=jnp.float32)
        h = (g / (1.0 + jnp.exp(-g))) * u
        out_ref[...] = lax.dot_general(
            h.astype(jnp.bfloat16), wd_ref[0].astype(jnp.bfloat16), nt,
            preferred_element_type=jnp.float32)


_plan = pl.pallas_call(
    _plan_body,
    out_shape=(
        jax.ShapeDtypeStruct((_S, 1), jnp.int32),
        jax.ShapeDtypeStruct((_NB, 1), jnp.int32),
        jax.ShapeDtypeStruct((_NB, 1), jnp.int32),
    ),
)

_experts = pl.pallas_call(
    _expert_body,
    grid_spec=pltpu.PrefetchScalarGridSpec(
        num_scalar_prefetch=2,
        grid=(_NB,),
        in_specs=[
            pl.BlockSpec((_T, _H), lambda j, be, bv: (j, 0)),
            pl.BlockSpec((1, _FF, _H), lambda j, be, bv: (be[j], 0, 0)),
            pl.BlockSpec((1, _FF, _H), lambda j, be, bv: (be[j], 0, 0)),
            pl.BlockSpec((1, _H, _FF), lambda j, be, bv: (be[j], 0, 0)),
        ],
        out_specs=pl.BlockSpec((_T, _H), lambda j, be, bv: (j, 0)),
    ),
    out_shape=jax.ShapeDtypeStruct((_PS, _H), jnp.float32),
)

@functools.cache
def _sc_kernels():
    mesh = plsc.VectorSubcoreMesh(core_axis_name="c", subcore_axis_name="s")
    scratch = [
        pltpu.VMEM((_CHUNK,), jnp.int32),
        pltpu.VMEM((_CHUNK, _H), jnp.float32),
        pltpu.SemaphoreType.DMA,
    ]

    @functools.partial(
        pl.kernel,
        out_type=jax.ShapeDtypeStruct((_PS, _H), jnp.float32),
        mesh=mesh,
        scratch_types=scratch,
    )
    def dispatch(x_hbm, pos_hbm, xs_hbm, idx_v, rows_v, sem):
        wid = lax.axis_index("s") * 2 + lax.axis_index("c")
        base = wid * _CHUNK
        pltpu.sync_copy(pos_hbm.at[pl.ds(base, _CHUNK)], idx_v)
        pltpu.sync_copy(x_hbm.at[pl.ds(base, _CHUNK)], rows_v)
        pltpu.async_copy(rows_v, xs_hbm.at[idx_v], sem).wait()

    @functools.partial(
        pl.kernel,
        out_type=jax.ShapeDtypeStruct((_S, _H), jnp.float32),
        mesh=mesh,
        scratch_types=scratch,
    )
    def combine(po_hbm, pos_hbm, out_hbm, idx_v, rows_v, sem):
        wid = lax.axis_index("s") * 2 + lax.axis_index("c")
        base = wid * _CHUNK
        pltpu.sync_copy(pos_hbm.at[pl.ds(base, _CHUNK)], idx_v)
        pltpu.async_copy(po_hbm.at[idx_v], rows_v, sem).wait()
        pltpu.sync_copy(rows_v, out_hbm.at[pl.ds(base, _CHUNK)])

    return dispatch, combine


@jax.jit
def kernel(x, gate_w, w_gate, w_up, w_down):
    b, s, h = x.shape
    x_flat = x.reshape(s, h)
    dispatch, combine = _sc_kernels()
    pos2, be2, bv2 = _plan(x_flat, gate_w)
    pos = pos2.reshape(s)
    xs = dispatch(x_flat, pos)
    po = _experts(be2.reshape(-1), bv2.reshape(-1), xs, w_gate, w_up, w_down)
    out = combine(po, pos)
    return out.reshape(b, s, h)
